# baseline (speedup 1.0000x reference)
import numpy as np

import jax
import jax.numpy as jnp
from jax import lax
from jax.experimental import pallas as pl
from jax.experimental.pallas import tpu as pltpu
from jax.experimental.pallas import tpu_sc as plsc

N = 10000
E = 320000
D = 128
K = 3
A = 1.0
B = 1.0
LO = -1.0
HI = 1.0

NC = 2
NS = 16
NW = NC * NS
CB = 128
CH = -(-E // (NW * CB))
EPW = CH * CB
EPAD = NW * EPW
NP = 10112
RPT = NP // NS

COEF1 = (A - B) / 2 - (A + B + 2) / 2 * ((LO + HI) / (HI - LO))
COEF2 = (A + B + 2) / (HI - LO)

ZSTARTS = (0, 128, 256, 384, 504)
ZCH = len(ZSTARTS)
_ZIDX_NP = (np.arange(NS)[:, None, None] * RPT
            + np.asarray(ZSTARTS)[None, :, None]
            + np.arange(CB)[None, None, :]).astype(np.int32)


def _jacobi_facs(L):
    coef_l = 2 * L * (L + A + B) * (2 * L - 2 + A + B)
    c11 = (2 * L + A + B - 1) * (2 * L + A + B) * (2 * L + A + B - 2)
    c12 = (2 * L + A + B - 1) * (A ** 2 - B ** 2)
    c2 = 2 * (L - 1 + A) * (L - 1 + B) * (2 * L + A + B)
    return c11 / coef_l, c12 / coef_l, c2 / coef_l


def _fill2d(ref, rows, width, value):
    def body(i, carry):
        for k in range(width // 16):
            ref[i, pl.ds(k * 16, 16)] = jnp.full((16,), value, jnp.float32)
        return carry
    lax.fori_loop(0, rows, body, 0)


_MESH = plsc.VectorSubcoreMesh(core_axis_name="c", subcore_axis_name="s")


def _deg_body(row_hbm, zidx_hbm, out_hbm, rowv, onesv, buf, zidxv, degsh, sem):
    c = lax.axis_index("c")
    s = lax.axis_index("s")
    wid = c * NS + s
    base = s * RPT
    _fill2d(buf, CB, D, 0.0)
    _fill2d(onesv, CB, D, 1.0)
    pltpu.sync_copy(zidx_hbm.at[s], zidxv)
    pltpu.sync_copy(row_hbm.at[wid], rowv)
    for k in range(ZCH):
        pltpu.sync_copy(buf, degsh.at[zidxv.at[k]])
    plsc.subcore_barrier()

    def body(j, carry):
        pltpu.sync_copy(onesv, degsh.at[rowv.at[j]], add=True)
        return carry

    lax.fori_loop(0, CH, body, 0)
    plsc.subcore_barrier()
    for k in range(ZCH):
        pltpu.async_copy(degsh.at[zidxv.at[k]], buf, sem).wait()
        pltpu.sync_copy(buf, out_hbm.at[c, pl.ds(base + ZSTARTS[k], CB)])


_deg_call = pl.kernel(
    _deg_body,
    out_type=jax.ShapeDtypeStruct((NC, NP, D), jnp.float32),
    mesh=_MESH,
    scratch_types=[
        pltpu.VMEM((CH, CB), jnp.int32),
        pltpu.VMEM((CB, D), jnp.float32),
        pltpu.VMEM((CB, D), jnp.float32),
        pltpu.VMEM((ZCH, CB), jnp.int32),
        pltpu.VMEM_SHARED((NP, D), jnp.float32),
        pltpu.SemaphoreType.DMA,
    ],
)


SBF = 8
CH0 = 80
CH1 = 80
NCHT = NS * (CH0 + CH1)
EPAD_F = NCHT * CB


def _spmm_body(u_hbm, col_hbm, row_hbm, zidx_hbm, out_hbm,
               colv, rowv, gbuf0, gbuf1, zidxv, ysh, sem0, sem1, sem2, sem3):
    c = lax.axis_index("c")
    s = lax.axis_index("s")
    base = s * RPT
    _fill2d(gbuf0, CB, D, 0.0)
    pltpu.sync_copy(zidx_hbm.at[s], zidxv)
    for k in range(ZCH):
        pltpu.sync_copy(gbuf0, ysh.at[zidxv.at[k]])
    plsc.subcore_barrier()

    gb = (gbuf0, gbuf1)
    gsems = (sem0, sem1)
    ssems = (sem2, sem3)

    def run_block(nchunks):
        dg = [None, None]
        dsc = [None, None]
        dg[0] = pltpu.async_copy(u_hbm.at[colv.at[0]], gbuf0, sem0)
        for k in range(nchunks):
            b = k % 2
            dg[b].wait()
            dsc[b] = pltpu.async_copy(gb[b], ysh.at[rowv.at[k]],
                                      ssems[b], add=True)
            if k + 1 < nchunks:
                b2 = (k + 1) % 2
                if dsc[b2] is not None:
                    dsc[b2].wait()
                    dsc[b2] = None
                dg[b2] = pltpu.async_copy(u_hbm.at[colv.at[k + 1]],
                                          gb[b2], gsems[b2])
        for b in range(2):
            if dsc[b] is not None:
                dsc[b].wait()

    def edge_loop(base_chunk, chx):
        nsb = chx // SBF
        tail = chx - nsb * SBF

        def sbody(sb, carry):
            st = base_chunk + sb * SBF
            pltpu.sync_copy(col_hbm.at[pl.ds(st, SBF)], colv)
            pltpu.sync_copy(row_hbm.at[pl.ds(st, SBF)], rowv)
            run_block(SBF)
            return carry

        lax.fori_loop(0, nsb, sbody, 0)
        if tail:
            st = base_chunk + nsb * SBF
            pltpu.sync_copy(col_hbm.at[pl.ds(st, tail)],
                            colv.at[pl.ds(0, tail)])
            pltpu.sync_copy(row_hbm.at[pl.ds(st, tail)],
                            rowv.at[pl.ds(0, tail)])
            run_block(tail)

    @pl.when(c == 0)
    def _():
        edge_loop(s * CH0, CH0)

    @pl.when(c == 1)
    def _():
        edge_loop(NS * CH0 + s * CH1, CH1)

    plsc.subcore_barrier()
    for k in range(ZCH):
        pltpu.async_copy(ysh.at[zidxv.at[k]], gbuf0, sem0).wait()
        pltpu.sync_copy(gbuf0, out_hbm.at[c, pl.ds(base + ZSTARTS[k], CB)])


_spmm_call = pl.kernel(
    _spmm_body,
    out_type=jax.ShapeDtypeStruct((NC, NP, D), jnp.float32),
    mesh=_MESH,
    scratch_types=[
        pltpu.VMEM((SBF, CB), jnp.int32),
        pltpu.VMEM((SBF, CB), jnp.int32),
        pltpu.VMEM((CB, D), jnp.float32),
        pltpu.VMEM((CB, D), jnp.float32),
        pltpu.VMEM((ZCH, CB), jnp.int32),
        pltpu.VMEM_SHARED((NP, D), jnp.float32),
        pltpu.SemaphoreType.DMA,
        pltpu.SemaphoreType.DMA,
        pltpu.SemaphoreType.DMA,
        pltpu.SemaphoreType.DMA,
    ],
)


def _prep_body(deg_ref, x_ref, dinv_ref, u0_ref):
    dt = deg_ref[0] + deg_ref[1]
    dt = jnp.where(dt < 0.5, dt + 1.0, dt)
    dv = lax.rsqrt(dt)
    dinv_ref[...] = dv
    u0_ref[...] = x_ref[...] * dv


_prep_call = pl.pallas_call(
    _prep_body,
    out_shape=(
        jax.ShapeDtypeStruct((N, 1), jnp.float32),
        jax.ShapeDtypeStruct((N, D), jnp.float32),
    ),
)


def _comb1_body(ap_ref, zp_ref, x0_ref, dinv_ref, x1_ref, u1_ref):
    a = jnp.tanh(ap_ref[...])
    a0 = a[0:1, 0:1]
    dv = dinv_ref[...]
    sx = (zp_ref[0][:N] + zp_ref[1][:N]) * dv
    x1 = a0 * (COEF1 * x0_ref[...] + COEF2 * sx)
    x1_ref[...] = x1
    u1_ref[...] = x1 * dv


_comb1_call = pl.pallas_call(
    _comb1_body,
    out_shape=(
        jax.ShapeDtypeStruct((N, D), jnp.float32),
        jax.ShapeDtypeStruct((N, D), jnp.float32),
    ),
)


def _comb2_body(ap_ref, zp_ref, x1_ref, x0_ref, dinv_ref, x2_ref, u2_ref):
    f11, f12, f2 = _jacobi_facs(2)
    a = jnp.tanh(ap_ref[...])
    a1 = a[1:2, 0:1]
    a0 = a[0:1, 0:1]
    t1 = a1 * f11
    t2 = a1 * f12
    t3 = a1 * a0 * f2
    t1_2 = t1 * (2.0 / (HI - LO))
    t2_2 = t1 * ((HI + LO) / (HI - LO)) + t2
    dv = dinv_ref[...]
    sx = (zp_ref[0][:N] + zp_ref[1][:N]) * dv
    x2 = t1_2 * sx - t2_2 * x1_ref[...] - t3 * x0_ref[...]
    x2_ref[...] = x2
    u2_ref[...] = x2 * dv


_comb2_call = pl.pallas_call(
    _comb2_body,
    out_shape=(
        jax.ShapeDtypeStruct((N, D), jnp.float32),
        jax.ShapeDtypeStruct((N, D), jnp.float32),
    ),
)


def _final_body(ap_ref, zp_ref, x2_ref, x1_ref, x0_ref, dinv_ref, wt_ref,
                b_ref, out_ref):
    f11, f12, f2 = _jacobi_facs(3)
    a = jnp.tanh(ap_ref[...])
    a2 = a[2:3, 0:1]
    a1 = a[1:2, 0:1]
    t1 = a2 * f11
    t2 = a2 * f12
    t3 = a2 * a1 * f2
    t1_2 = t1 * (2.0 / (HI - LO))
    t2_2 = t1 * ((HI + LO) / (HI - LO)) + t2
    dv = dinv_ref[...]
    sx = (zp_ref[0][:N] + zp_ref[1][:N]) * dv
    x2v = x2_ref[...]
    x1v = x1_ref[...]
    x3 = t1_2 * sx - t2_2 * x2v - t3 * x1v
    acc = jnp.dot(x0_ref[...], wt_ref[0:D],
                  preferred_element_type=jnp.float32)
    acc = acc + jnp.dot(x1v, wt_ref[D:2 * D],
                        preferred_element_type=jnp.float32)
    acc = acc + jnp.dot(x2v, wt_ref[2 * D:3 * D],
                        preferred_element_type=jnp.float32)
    acc = acc + jnp.dot(x3, wt_ref[3 * D:4 * D],
                        preferred_element_type=jnp.float32)
    out_ref[...] = acc + b_ref[...]


_final_call = pl.pallas_call(
    _final_body,
    out_shape=jax.ShapeDtypeStruct((N, D), jnp.float32),
)


def kernel(x, edge_index, alphas_param, W, bias):
    row = edge_index[0]
    col = edge_index[1]
    row_pad = jnp.concatenate([row, jnp.full((EPAD_F - E,), N, jnp.int32)])
    col_pad = jnp.concatenate([col, jnp.zeros((EPAD_F - E,), jnp.int32)])
    row_p = row_pad[:EPAD].reshape(NW, CH, CB)
    row_f = row_pad.reshape(NCHT, CB)
    col_f = col_pad.reshape(NCHT, CB)
    zidx = jnp.asarray(_ZIDX_NP)
    ap2 = jnp.broadcast_to(alphas_param[:, None], (K + 1, D))
    wt = W.T
    b2 = bias[None, :]

    degp = _deg_call(row_p, zidx)[:, :N, 0:1]
    dinv, u0 = _prep_call(degp, x)
    zp1 = _spmm_call(u0, col_f, row_f, zidx)
    x1, u1 = _comb1_call(ap2, zp1, x, dinv)
    zp2 = _spmm_call(u1, col_f, row_f, zidx)
    x2, u2 = _comb2_call(ap2, zp2, x1, x, dinv)
    zp3 = _spmm_call(u2, col_f, row_f, zidx)
    return _final_call(ap2, zp3, x2, x1, x, dinv, wt, b2)


# speedup vs baseline: 1.5335x; 1.5335x over previous
import numpy as np

import jax
import jax.numpy as jnp
from jax import lax
from jax.experimental import pallas as pl
from jax.experimental.pallas import tpu as pltpu
from jax.experimental.pallas import tpu_sc as plsc

N = 10000
E = 320000
D = 128
K = 3
A = 1.0
B = 1.0
LO = -1.0
HI = 1.0

NC = 2
NS = 16
NW = NC * NS
CB = 128
CH = -(-E // (NW * CB))
EPW = CH * CB
EPAD = NW * EPW
NP = 10112
RPT = NP // NS

COEF1 = (A - B) / 2 - (A + B + 2) / 2 * ((LO + HI) / (HI - LO))
COEF2 = (A + B + 2) / (HI - LO)

ZSTARTS = (0, 128, 256, 384, 504)
ZCH = len(ZSTARTS)
_ZIDX_NP = (np.arange(NS)[:, None, None] * RPT
            + np.asarray(ZSTARTS)[None, :, None]
            + np.arange(CB)[None, None, :]).astype(np.int32)


def _jacobi_facs(L):
    coef_l = 2 * L * (L + A + B) * (2 * L - 2 + A + B)
    c11 = (2 * L + A + B - 1) * (2 * L + A + B) * (2 * L + A + B - 2)
    c12 = (2 * L + A + B - 1) * (A ** 2 - B ** 2)
    c2 = 2 * (L - 1 + A) * (L - 1 + B) * (2 * L + A + B)
    return c11 / coef_l, c12 / coef_l, c2 / coef_l


def _fill2d(ref, rows, width, value):
    def body(i, carry):
        for k in range(width // 16):
            ref[i, pl.ds(k * 16, 16)] = jnp.full((16,), value, jnp.float32)
        return carry
    lax.fori_loop(0, rows, body, 0)


_MESH = plsc.VectorSubcoreMesh(core_axis_name="c", subcore_axis_name="s")


def _deg_body(row_hbm, zidx_hbm, out_hbm, rowv, onesv, buf, zidxv, degsh, sem):
    c = lax.axis_index("c")
    s = lax.axis_index("s")
    wid = c * NS + s
    base = s * RPT
    _fill2d(buf, CB, D, 0.0)
    _fill2d(onesv, CB, D, 1.0)
    pltpu.sync_copy(zidx_hbm.at[s], zidxv)
    pltpu.sync_copy(row_hbm.at[wid], rowv)
    for k in range(ZCH):
        pltpu.sync_copy(buf, degsh.at[zidxv.at[k]])
    plsc.subcore_barrier()

    def body(j, carry):
        pltpu.sync_copy(onesv, degsh.at[rowv.at[j]], add=True)
        return carry

    lax.fori_loop(0, CH, body, 0)
    plsc.subcore_barrier()
    for k in range(ZCH):
        pltpu.async_copy(degsh.at[zidxv.at[k]], buf, sem).wait()
        pltpu.sync_copy(buf, out_hbm.at[c, pl.ds(base + ZSTARTS[k], CB)])


_deg_call = pl.kernel(
    _deg_body,
    out_type=jax.ShapeDtypeStruct((NC, NP, D), jnp.float32),
    mesh=_MESH,
    scratch_types=[
        pltpu.VMEM((CH, CB), jnp.int32),
        pltpu.VMEM((CB, D), jnp.float32),
        pltpu.VMEM((CB, D), jnp.float32),
        pltpu.VMEM((ZCH, CB), jnp.int32),
        pltpu.VMEM_SHARED((NP, D), jnp.float32),
        pltpu.SemaphoreType.DMA,
    ],
)


SBF = 8
NSB = CH // SBF
TAIL = CH - NSB * SBF


def _spmm_body(u_hbm, col_hbm, row_hbm, zidx_hbm, out_hbm,
               colv, rowv, gbuf0, gbuf1, zidxv, ysh, sem0, sem1, sem2, sem3):
    c = lax.axis_index("c")
    s = lax.axis_index("s")
    wid = c * NS + s
    base = s * RPT
    _fill2d(gbuf0, CB, D, 0.0)
    pltpu.sync_copy(zidx_hbm.at[s], zidxv)
    for k in range(ZCH):
        pltpu.sync_copy(gbuf0, ysh.at[zidxv.at[k]])
    plsc.subcore_barrier()

    gb = (gbuf0, gbuf1)
    gsems = (sem0, sem1)
    ssems = (sem2, sem3)

    def run_block(nchunks):
        dg = [None, None]
        dsc = [None, None]
        dg[0] = pltpu.async_copy(u_hbm.at[colv.at[0]], gbuf0, sem0)
        for k in range(nchunks):
            b = k % 2
            dg[b].wait()
            dsc[b] = pltpu.async_copy(gb[b], ysh.at[rowv.at[k]],
                                      ssems[b], add=True)
            if k + 1 < nchunks:
                b2 = (k + 1) % 2
                if dsc[b2] is not None:
                    dsc[b2].wait()
                    dsc[b2] = None
                dg[b2] = pltpu.async_copy(u_hbm.at[colv.at[k + 1]],
                                          gb[b2], gsems[b2])
        for b in range(2):
            if dsc[b] is not None:
                dsc[b].wait()

    def sbody(sb, carry):
        pltpu.sync_copy(col_hbm.at[wid, pl.ds(sb * SBF, SBF)], colv)
        pltpu.sync_copy(row_hbm.at[wid, pl.ds(sb * SBF, SBF)], rowv)
        run_block(SBF)
        return carry

    lax.fori_loop(0, NSB, sbody, 0)
    pltpu.sync_copy(col_hbm.at[wid, pl.ds(NSB * SBF, TAIL)],
                    colv.at[pl.ds(0, TAIL)])
    pltpu.sync_copy(row_hbm.at[wid, pl.ds(NSB * SBF, TAIL)],
                    rowv.at[pl.ds(0, TAIL)])
    run_block(TAIL)
    plsc.subcore_barrier()
    for k in range(ZCH):
        pltpu.async_copy(ysh.at[zidxv.at[k]], gbuf0, sem0).wait()
        pltpu.sync_copy(gbuf0, out_hbm.at[c, pl.ds(base + ZSTARTS[k], CB)])


_spmm_call = pl.kernel(
    _spmm_body,
    out_type=jax.ShapeDtypeStruct((NC, NP, D), jnp.float32),
    mesh=_MESH,
    scratch_types=[
        pltpu.VMEM((SBF, CB), jnp.int32),
        pltpu.VMEM((SBF, CB), jnp.int32),
        pltpu.VMEM((CB, D), jnp.float32),
        pltpu.VMEM((CB, D), jnp.float32),
        pltpu.VMEM((ZCH, CB), jnp.int32),
        pltpu.VMEM_SHARED((NP, D), jnp.float32),
        pltpu.SemaphoreType.DMA,
        pltpu.SemaphoreType.DMA,
        pltpu.SemaphoreType.DMA,
        pltpu.SemaphoreType.DMA,
    ],
)


def _prep_body(deg_ref, x_ref, dinv_ref, u0_ref):
    dt = deg_ref[0] + deg_ref[1]
    dt = jnp.where(dt < 0.5, dt + 1.0, dt)
    dv = lax.rsqrt(dt)
    dinv_ref[...] = dv
    u0_ref[...] = x_ref[...] * dv


_prep_call = pl.pallas_call(
    _prep_body,
    out_shape=(
        jax.ShapeDtypeStruct((N, 1), jnp.float32),
        jax.ShapeDtypeStruct((N, D), jnp.float32),
    ),
)


def _comb1_body(ap_ref, zp_ref, x0_ref, dinv_ref, x1_ref, u1_ref):
    a = jnp.tanh(ap_ref[...])
    a0 = a[0:1, 0:1]
    dv = dinv_ref[...]
    sx = (zp_ref[0][:N] + zp_ref[1][:N]) * dv
    x1 = a0 * (COEF1 * x0_ref[...] + COEF2 * sx)
    x1_ref[...] = x1
    u1_ref[...] = x1 * dv


_comb1_call = pl.pallas_call(
    _comb1_body,
    out_shape=(
        jax.ShapeDtypeStruct((N, D), jnp.float32),
        jax.ShapeDtypeStruct((N, D), jnp.float32),
    ),
)


def _comb2_body(ap_ref, zp_ref, x1_ref, x0_ref, dinv_ref, x2_ref, u2_ref):
    f11, f12, f2 = _jacobi_facs(2)
    a = jnp.tanh(ap_ref[...])
    a1 = a[1:2, 0:1]
    a0 = a[0:1, 0:1]
    t1 = a1 * f11
    t2 = a1 * f12
    t3 = a1 * a0 * f2
    t1_2 = t1 * (2.0 / (HI - LO))
    t2_2 = t1 * ((HI + LO) / (HI - LO)) + t2
    dv = dinv_ref[...]
    sx = (zp_ref[0][:N] + zp_ref[1][:N]) * dv
    x2 = t1_2 * sx - t2_2 * x1_ref[...] - t3 * x0_ref[...]
    x2_ref[...] = x2
    u2_ref[...] = x2 * dv


_comb2_call = pl.pallas_call(
    _comb2_body,
    out_shape=(
        jax.ShapeDtypeStruct((N, D), jnp.float32),
        jax.ShapeDtypeStruct((N, D), jnp.float32),
    ),
)


def _final_body(ap_ref, zp_ref, x2_ref, x1_ref, x0_ref, dinv_ref, wt_ref,
                b_ref, out_ref):
    f11, f12, f2 = _jacobi_facs(3)
    a = jnp.tanh(ap_ref[...])
    a2 = a[2:3, 0:1]
    a1 = a[1:2, 0:1]
    t1 = a2 * f11
    t2 = a2 * f12
    t3 = a2 * a1 * f2
    t1_2 = t1 * (2.0 / (HI - LO))
    t2_2 = t1 * ((HI + LO) / (HI - LO)) + t2
    dv = dinv_ref[...]
    sx = (zp_ref[0][:N] + zp_ref[1][:N]) * dv
    x2v = x2_ref[...]
    x1v = x1_ref[...]
    x3 = t1_2 * sx - t2_2 * x2v - t3 * x1v
    acc = jnp.dot(x0_ref[...], wt_ref[0:D],
                  preferred_element_type=jnp.float32)
    acc = acc + jnp.dot(x1v, wt_ref[D:2 * D],
                        preferred_element_type=jnp.float32)
    acc = acc + jnp.dot(x2v, wt_ref[2 * D:3 * D],
                        preferred_element_type=jnp.float32)
    acc = acc + jnp.dot(x3, wt_ref[3 * D:4 * D],
                        preferred_element_type=jnp.float32)
    out_ref[...] = acc + b_ref[...]


_final_call = pl.pallas_call(
    _final_body,
    out_shape=jax.ShapeDtypeStruct((N, D), jnp.float32),
)


def kernel(x, edge_index, alphas_param, W, bias):
    row = edge_index[0]
    col = edge_index[1]
    row_pad = jnp.concatenate([row, jnp.full((EPAD - E,), N, jnp.int32)])
    col_pad = jnp.concatenate([col, jnp.zeros((EPAD - E,), jnp.int32)])
    row_p = row_pad.reshape(NW, CH, CB)
    row_f = row_p
    col_f = col_pad.reshape(NW, CH, CB)
    zidx = jnp.asarray(_ZIDX_NP)
    ap2 = jnp.broadcast_to(alphas_param[:, None], (K + 1, D))
    wt = W.T
    b2 = bias[None, :]

    degp = _deg_call(row_p, zidx)[:, :N, 0:1]
    dinv, u0 = _prep_call(degp, x)
    zp1 = _spmm_call(u0, col_f, row_f, zidx)
    x1, u1 = _comb1_call(ap2, zp1, x, dinv)
    zp2 = _spmm_call(u1, col_f, row_f, zidx)
    x2, u2 = _comb2_call(ap2, zp2, x1, x, dinv)
    zp3 = _spmm_call(u2, col_f, row_f, zidx)
    return _final_call(ap2, zp3, x2, x1, x, dinv, wt, b2)
